# baseline (speedup 1.0000x reference)
import functools

import jax
import jax.numpy as jnp
from jax import lax
from jax.experimental import pallas as pl
from jax.experimental.pallas import tpu as pltpu
from jax.experimental.pallas import tpu_sc as plsc

D = 128
B = 4096
QL = 20
AL = 50
NW = 32
IW = B // NW
QIC = 8
AIC = 4
QNC = IW // QIC
ANC = IW // AIC
NB = 4
BUFROWS = AIC * AL


def _splits(n):
    out = []
    while n > 0:
        s = min(n, 128)
        out.append(s)
        n -= s
    return out


def _sc_gather_one(idx, table, nchunks, ipc, l):
    mesh = plsc.VectorSubcoreMesh(core_axis_name="c", subcore_axis_name="s")
    nidx = ipc * l
    subs = _splits(nidx)

    @functools.partial(
        pl.kernel,
        mesh=mesh,
        compiler_params=pltpu.CompilerParams(use_tc_tiling_on_sc=True),
        out_type=jax.ShapeDtypeStruct((B, l, D), jnp.float32),
        scratch_types=(
            [pltpu.VMEM((IW * l,), jnp.int32)]
            + [pltpu.VMEM((nidx, D), jnp.float32) for _ in range(NB)]
            + [pltpu.SemaphoreType.DMA for _ in range(2 * NB)]
        ),
    )
    def k(idx_hbm, table_hbm, out_hbm, idx_v, *scratch):
        rows = scratch[:NB]
        gsem = scratch[NB:2 * NB]
        ssem = scratch[2 * NB:]
        wid = lax.axis_index("s") * 2 + lax.axis_index("c")
        item0 = wid * IW

        pltpu.sync_copy(idx_hbm.at[pl.ds(wid * IW * l, IW * l)], idx_v)

        def drain_stores(b):
            pltpu.make_async_copy(
                rows[b], table_hbm.at[pl.ds(0, nidx)], ssem[b]).wait()

        def body(t, carry):
            descs = []
            for b in range(NB):
                c = t * NB + b

                @pl.when(t > 0)
                def _():
                    drain_stores(b)

                off = 0
                for s in subs:
                    descs.append(pltpu.async_copy(
                        table_hbm.at[idx_v.at[pl.ds(c * nidx + off, s)]],
                        rows[b].at[pl.ds(off, s)], gsem[b]))
                    off += s
            i = 0
            for b in range(NB):
                for _s in subs:
                    descs[i].wait()
                    i += 1
                c = t * NB + b
                for j in range(ipc):
                    pltpu.async_copy(
                        rows[b].at[pl.ds(j * l, l)],
                        out_hbm.at[item0 + c * ipc + j], ssem[b])
            return carry

        lax.fori_loop(0, nchunks // NB, body, 0)
        for b in range(NB):
            drain_stores(b)

    return k(idx, table)


def _masks(iq, ia):
    def body(q_ref, a_ref, mq_ref, ma_ref):
        for ref, out in ((q_ref, mq_ref), (a_ref, ma_ref)):
            x = ref[...].astype(jnp.float32)
            m = jnp.max(x, axis=1, keepdims=True)
            out[...] = jnp.ceil(x / m)

    nb = 8
    bb = B // nb
    return pl.pallas_call(
        body,
        grid=(nb,),
        in_specs=[
            pl.BlockSpec((bb, QL), lambda i: (i, 0)),
            pl.BlockSpec((bb, AL), lambda i: (i, 0)),
        ],
        out_specs=[
            pl.BlockSpec((bb, QL), lambda i: (i, 0)),
            pl.BlockSpec((bb, AL), lambda i: (i, 0)),
        ],
        out_shape=[
            jax.ShapeDtypeStruct((B, QL), jnp.float32),
            jax.ShapeDtypeStruct((B, AL), jnp.float32),
        ],
    )(iq, ia)


def kernel(input_question, input_answer, embeddings):
    eq = _sc_gather_one(
        input_question.reshape(-1), embeddings, QNC, QIC, QL)
    ea = _sc_gather_one(
        input_answer.reshape(-1), embeddings, ANC, AIC, AL)
    mq, ma = _masks(input_question, input_answer)
    return eq, ea, mq, ma


# speedup vs baseline: 1.7715x; 1.7715x over previous
import functools

import jax
import jax.numpy as jnp
from jax import lax
from jax.experimental import pallas as pl
from jax.experimental.pallas import tpu as pltpu
from jax.experimental.pallas import tpu_sc as plsc

D = 128
B = 4096
QL = 20
AL = 50
NQ = B * QL
NA = B * AL
NW = 32
CH = 128
QCH = NQ // (NW * CH)
ACH = NA // (NW * CH)
NB = 5


def _sc_gather(idx_q, idx_a, table):
    mesh = plsc.VectorSubcoreMesh(core_axis_name="c", subcore_axis_name="s")

    @functools.partial(
        pl.kernel,
        mesh=mesh,
        out_type=[
            jax.ShapeDtypeStruct((NQ, D), jnp.float32),
            jax.ShapeDtypeStruct((NA, D), jnp.float32),
        ],
        scratch_types=(
            [pltpu.VMEM((QCH * CH,), jnp.int32),
             pltpu.VMEM((ACH * CH,), jnp.int32)]
            + [pltpu.VMEM((CH, D), jnp.float32) for _ in range(NB)]
            + [pltpu.SemaphoreType.DMA for _ in range(2 * NB)]
        ),
    )
    def k(idx_q_hbm, idx_a_hbm, table_hbm, out_q_hbm, out_a_hbm,
          idx_q_v, idx_a_v, *scratch):
        rows = scratch[:NB]
        gsem = scratch[NB:2 * NB]
        ssem = scratch[2 * NB:]
        wid = lax.axis_index("s") * 2 + lax.axis_index("c")

        pltpu.sync_copy(idx_q_hbm.at[pl.ds(wid * QCH * CH, QCH * CH)], idx_q_v)
        pltpu.sync_copy(idx_a_hbm.at[pl.ds(wid * ACH * CH, ACH * CH)], idx_a_v)

        def run(idx_v, out_hbm, nchunks, first):
            base = wid * nchunks * CH

            def body(t, carry):
                descs = []
                for b in range(NB):
                    c = t * NB + b
                    if not first:
                        pltpu.make_async_copy(
                            rows[b], out_hbm.at[pl.ds(0, CH)], ssem[b]).wait()
                    else:
                        @pl.when(t > 0)
                        def _():
                            pltpu.make_async_copy(
                                rows[b], out_hbm.at[pl.ds(0, CH)],
                                ssem[b]).wait()
                    descs.append(pltpu.async_copy(
                        table_hbm.at[idx_v.at[pl.ds(c * CH, CH)]],
                        rows[b], gsem[b]))
                for b in range(NB):
                    descs[b].wait()
                    pltpu.async_copy(
                        rows[b],
                        out_hbm.at[pl.ds(base + (t * NB + b) * CH, CH)],
                        ssem[b])
                return carry

            lax.fori_loop(0, nchunks // NB, body, 0)

        run(idx_q_v, out_q_hbm, QCH, first=True)
        run(idx_a_v, out_a_hbm, ACH, first=False)
        for b in range(NB):
            pltpu.make_async_copy(
                rows[b], out_a_hbm.at[pl.ds(0, CH)], ssem[b]).wait()

    return k(idx_q, idx_a, table)


def _masks(iq, ia):
    def body(q_ref, a_ref, mq_ref, ma_ref):
        for ref, out in ((q_ref, mq_ref), (a_ref, ma_ref)):
            x = ref[...].astype(jnp.float32)
            m = jnp.max(x, axis=1, keepdims=True)
            out[...] = jnp.ceil(x / m)

    nb = 8
    bb = B // nb
    return pl.pallas_call(
        body,
        grid=(nb,),
        in_specs=[
            pl.BlockSpec((bb, QL), lambda i: (i, 0)),
            pl.BlockSpec((bb, AL), lambda i: (i, 0)),
        ],
        out_specs=[
            pl.BlockSpec((bb, QL), lambda i: (i, 0)),
            pl.BlockSpec((bb, AL), lambda i: (i, 0)),
        ],
        out_shape=[
            jax.ShapeDtypeStruct((B, QL), jnp.float32),
            jax.ShapeDtypeStruct((B, AL), jnp.float32),
        ],
    )(iq, ia)


def kernel(input_question, input_answer, embeddings):
    iq_t = input_question.T.reshape(-1)
    ia_t = input_answer.T.reshape(-1)
    eq2, ea2 = _sc_gather(iq_t, ia_t, embeddings)
    eq = eq2.reshape(QL, B, D).transpose(1, 0, 2)
    ea = ea2.reshape(AL, B, D).transpose(1, 0, 2)
    mq, ma = _masks(input_question, input_answer)
    return eq, ea, mq, ma


# speedup vs baseline: 1.8379x; 1.0375x over previous
import functools

import jax
import jax.numpy as jnp
from jax import lax
from jax.experimental import pallas as pl
from jax.experimental.pallas import tpu as pltpu
from jax.experimental.pallas import tpu_sc as plsc

D = 128
B = 4096
QL = 20
AL = 50
NQ = B * QL
NA = B * AL
NW = 32
CH = 128
QCH = NQ // (NW * CH)
ACH = NA // (NW * CH)
NB = 5


def _sc_gather(idx_q, idx_a, table):
    mesh = plsc.VectorSubcoreMesh(core_axis_name="c", subcore_axis_name="s")

    @functools.partial(
        pl.kernel,
        mesh=mesh,
        out_type=[
            jax.ShapeDtypeStruct((NQ, D), jnp.float32),
            jax.ShapeDtypeStruct((NA, D), jnp.float32),
        ],
        scratch_types=(
            [pltpu.VMEM((QCH * CH,), jnp.int32),
             pltpu.VMEM((ACH * CH,), jnp.int32)]
            + [pltpu.VMEM((CH, D), jnp.float32) for _ in range(NB)]
            + [pltpu.SemaphoreType.DMA for _ in range(2 * NB)]
        ),
    )
    def k(idx_q_hbm, idx_a_hbm, table_hbm, out_q_hbm, out_a_hbm,
          idx_q_v, idx_a_v, *scratch):
        rows = scratch[:NB]
        gsem = scratch[NB:2 * NB]
        ssem = scratch[2 * NB:]
        wid = lax.axis_index("s") * 2 + lax.axis_index("c")

        pltpu.sync_copy(idx_q_hbm.at[pl.ds(wid * QCH * CH, QCH * CH)], idx_q_v)
        pltpu.sync_copy(idx_a_hbm.at[pl.ds(wid * ACH * CH, ACH * CH)], idx_a_v)

        def run(idx_v, out_hbm, nchunks, first):
            base = wid * nchunks * CH

            def body(t, carry):
                descs = []
                for b in range(NB):
                    c = t * NB + b
                    if not first:
                        pltpu.make_async_copy(
                            rows[b], out_hbm.at[pl.ds(0, CH)], ssem[b]).wait()
                    else:
                        @pl.when(t > 0)
                        def _():
                            pltpu.make_async_copy(
                                rows[b], out_hbm.at[pl.ds(0, CH)],
                                ssem[b]).wait()
                    descs.append(pltpu.async_copy(
                        table_hbm.at[idx_v.at[pl.ds(c * CH, CH)]],
                        rows[b], gsem[b]))
                for b in range(NB):
                    descs[b].wait()
                    pltpu.async_copy(
                        rows[b],
                        out_hbm.at[pl.ds(base + (t * NB + b) * CH, CH)],
                        ssem[b])
                return carry

            lax.fori_loop(0, nchunks // NB, body, 0)

        run(idx_q_v, out_q_hbm, QCH, first=True)
        run(idx_a_v, out_a_hbm, ACH, first=False)
        for b in range(NB):
            pltpu.make_async_copy(
                rows[b], out_a_hbm.at[pl.ds(0, CH)], ssem[b]).wait()

    return k(idx_q, idx_a, table)


def _masks(iq_t, ia_t):
    def body(q_ref, a_ref, mq_ref, ma_ref):
        for ref, out in ((q_ref, mq_ref), (a_ref, ma_ref)):
            x = ref[...].astype(jnp.float32)
            m = jnp.max(x, axis=0, keepdims=True)
            out[...] = jnp.ceil(x / m)

    nb = 8
    bb = B // nb
    return pl.pallas_call(
        body,
        grid=(nb,),
        in_specs=[
            pl.BlockSpec((QL, bb), lambda i: (0, i)),
            pl.BlockSpec((AL, bb), lambda i: (0, i)),
        ],
        out_specs=[
            pl.BlockSpec((QL, bb), lambda i: (0, i)),
            pl.BlockSpec((AL, bb), lambda i: (0, i)),
        ],
        out_shape=[
            jax.ShapeDtypeStruct((QL, B), jnp.float32),
            jax.ShapeDtypeStruct((AL, B), jnp.float32),
        ],
    )(iq_t, ia_t)


def kernel(input_question, input_answer, embeddings):
    iq_t = input_question.T
    ia_t = input_answer.T
    eq2, ea2 = _sc_gather(iq_t.reshape(-1), ia_t.reshape(-1), embeddings)
    eq = eq2.reshape(QL, B, D).transpose(1, 0, 2)
    ea = ea2.reshape(AL, B, D).transpose(1, 0, 2)
    mq_t, ma_t = _masks(iq_t, ia_t)
    return eq, ea, mq_t.T, ma_t.T
